# Initial kernel scaffold; baseline (speedup 1.0000x reference)
import jax
import jax.numpy as jnp
from jax.experimental import pallas as pl


def kernel(x, edge_index, W_init, b_init, W_layers, W_final, b_final):
    raise NotImplementedError("write your pallas kernel here")


# speedup vs baseline: 21.1104x; 21.1104x over previous
import functools
import math

import jax
import jax.numpy as jnp
from jax import lax
from jax.experimental import pallas as pl
from jax.experimental.pallas import tpu as pltpu
from jax.experimental.pallas import tpu_sc as plsc

_ALPHA = 0.1
_BETA_BASE = 0.5

_NC = 2
_NS = 16
_NW = _NC * _NS
_K = 40
_NB = 4


def _sc_mesh():
    return plsc.VectorSubcoreMesh(
        core_axis_name="c", subcore_axis_name="s",
        num_cores=_NC, num_subcores=_NS)


def _row_split(n):
    br = 8 * (n // (8 * _NS))
    tail = n - br * _NS
    return br, tail


def _deg_parts(dst3, zeros16, n):
    nw, nch, k = dst3.shape
    br, tail = _row_split(n)

    @functools.partial(
        pl.kernel,
        out_type=jax.ShapeDtypeStruct((_NC, n, 16), jnp.float32),
        mesh=_sc_mesh(),
        scratch_types=[
            pltpu.VMEM((nch, k), jnp.int32),
            pltpu.VMEM((k, 16), jnp.float32),
            pltpu.VMEM_SHARED((n, 16), jnp.float32),
        ],
    )
    def kern(dst_hbm, zer_hbm, out_hbm, didx, ones_v, acc):
        cid = lax.axis_index("c")
        sid = lax.axis_index("s")
        wid = sid * _NC + cid
        pltpu.sync_copy(dst_hbm.at[wid], didx)
        onehot = jnp.where(lax.iota(jnp.int32, 16) == 0, 1.0, 0.0)

        @pl.loop(0, k)
        def _(r):
            ones_v[r, :] = onehot

        pltpu.sync_copy(zer_hbm.at[pl.ds(0, br)], acc.at[pl.ds(sid * br, br)])

        @pl.when(sid == _NS - 1)
        def _():
            pltpu.sync_copy(zer_hbm.at[pl.ds(0, tail)],
                            acc.at[pl.ds(_NS * br, tail)])

        plsc.subcore_barrier()

        @pl.loop(0, nch)
        def _(j):
            pltpu.sync_copy(ones_v, acc.at[didx.at[j]], add=True)

        plsc.subcore_barrier()
        pltpu.sync_copy(acc.at[pl.ds(sid * br, br)],
                        out_hbm.at[cid, pl.ds(sid * br, br)])

        @pl.when(sid == _NS - 1)
        def _():
            pltpu.sync_copy(acc.at[pl.ds(_NS * br, tail)],
                            out_hbm.at[cid, pl.ds(_NS * br, tail)])

    return kern(dst3, zeros16)


def _sc_scatter(hp, src2, dst3, zerosf):
    n, f = hp.shape
    ew = src2.shape[1]
    k = _K
    nch = ew // k
    br, tail = _row_split(n)

    nb = _NB
    assert nb % 2 == 0 and nch % 2 == 0
    nmain = (nch // nb) * nb

    @functools.partial(
        pl.kernel,
        out_type=jax.ShapeDtypeStruct((_NC, n, f), jnp.float32),
        mesh=_sc_mesh(),
        scratch_types=[
            pltpu.VMEM((ew,), jnp.int32),
            pltpu.VMEM((ew // 80, 80), jnp.int32),
            pltpu.VMEM((nb, k, f), jnp.float32),
            pltpu.VMEM_SHARED((n, f), jnp.float32),
        ] + [pltpu.SemaphoreType.DMA] * (2 * nb + 2),
    )
    def kern(hp_hbm, src_hbm, dst_hbm, zer_hbm, out_hbm, sidx, didx, rows, acc,
             *sems):
        gsem, ssem = sems[:nb], sems[nb:2 * nb]
        isem1, isem2 = sems[2 * nb], sems[2 * nb + 1]
        cid = lax.axis_index("c")
        sid = lax.axis_index("s")
        wid = sid * _NC + cid
        c1 = pltpu.async_copy(src_hbm.at[wid], sidx, isem1)
        c2 = pltpu.async_copy(dst_hbm.at[wid], didx, isem2)
        c3 = pltpu.async_copy(zer_hbm.at[pl.ds(0, br)],
                              acc.at[pl.ds(sid * br, br)], isem2)
        c1.wait()

        def dslice(j, b):
            return didx.at[j // 2, pl.ds((b % 2) * k, k)]

        for b in range(nb):
            pltpu.async_copy(hp_hbm.at[sidx.at[pl.ds(b * k, k)]], rows.at[b],
                             gsem[b])

        @pl.when(sid == _NS - 1)
        def _():
            pltpu.sync_copy(zer_hbm.at[pl.ds(0, tail)],
                            acc.at[pl.ds(_NS * br, tail)])

        c2.wait()
        c3.wait()
        plsc.subcore_barrier()

        @pl.loop(0, nmain, step=nb)
        def _(j0):
            for b in range(nb):
                j = j0 + b
                pltpu.make_async_copy(hp_hbm.at[sidx.at[pl.ds(j * k, k)]],
                                      rows.at[b], gsem[b]).wait()
                pltpu.async_copy(rows.at[b], acc.at[dslice(j, b)], ssem[b],
                                 add=True)
            for b in range(nb):
                j = j0 + b
                pltpu.make_async_copy(rows.at[b], acc.at[dslice(j, b)],
                                      ssem[b]).wait()
                jn = j + nb

                @pl.when(jn < nch)
                def _():
                    pltpu.async_copy(hp_hbm.at[sidx.at[pl.ds(jn * k, k)]],
                                     rows.at[b], gsem[b])

        for j in range(nmain, nch):
            b = j % nb
            pltpu.make_async_copy(hp_hbm.at[sidx.at[pl.ds(j * k, k)]],
                                  rows.at[b], gsem[b]).wait()
            pltpu.sync_copy(rows.at[b], acc.at[dslice(j, b)], add=True)

        plsc.subcore_barrier()
        pltpu.sync_copy(acc.at[pl.ds(sid * br, br)],
                        out_hbm.at[cid, pl.ds(sid * br, br)])

        @pl.when(sid == _NS - 1)
        def _():
            pltpu.sync_copy(acc.at[pl.ds(_NS * br, tail)],
                            out_hbm.at[cid, pl.ds(_NS * br, tail)])

    return kern(hp, src2, dst3, zerosf)


def _tc_init(x, w, b2, degp0, degp1, block_rows):
    n, fin = x.shape
    fh = w.shape[1]

    def body(x_ref, w_ref, b_ref, d0_ref, d1_ref, h0_ref, hp_ref, dinv_ref):
        deg = d0_ref[:, 0:1] + d1_ref[:, 0:1] + 1.0
        dinv = lax.rsqrt(deg)
        h = jnp.dot(x_ref[...], w_ref[...],
                    preferred_element_type=jnp.float32,
                    precision=lax.Precision.HIGHEST)
        h = jnp.maximum(h + b_ref[...], 0.0)
        h0_ref[...] = h
        hp_ref[...] = dinv * h
        dinv_ref[...] = dinv

    r = block_rows
    return pl.pallas_call(
        body,
        grid=(n // r,),
        in_specs=[
            pl.BlockSpec((r, fin), lambda i: (i, 0)),
            pl.BlockSpec((fin, fh), lambda i: (0, 0)),
            pl.BlockSpec((1, fh), lambda i: (0, 0)),
            pl.BlockSpec((r, 16), lambda i: (i, 0)),
            pl.BlockSpec((r, 16), lambda i: (i, 0)),
        ],
        out_specs=[
            pl.BlockSpec((r, fh), lambda i: (i, 0)),
            pl.BlockSpec((r, fh), lambda i: (i, 0)),
            pl.BlockSpec((r, 1), lambda i: (i, 0)),
        ],
        out_shape=[
            jax.ShapeDtypeStruct((n, fh), jnp.float32),
            jax.ShapeDtypeStruct((n, fh), jnp.float32),
            jax.ShapeDtypeStruct((n, 1), jnp.float32),
        ],
    )(x, w, b2, degp0, degp1)


def _tc_layer(s0, s1, hp, h0, dinv, w_eff, block_rows):
    n, fh = hp.shape

    def body(s0_ref, s1_ref, hp_ref, h0_ref, dinv_ref, w_ref, out_ref):
        dinv = dinv_ref[...]
        sup = ((1.0 - _ALPHA) * dinv * (s0_ref[...] + s1_ref[...] + hp_ref[...])
               + _ALPHA * h0_ref[...])
        h = jnp.dot(sup, w_ref[...], preferred_element_type=jnp.float32,
                    precision=lax.Precision.HIGHEST)
        out_ref[...] = dinv * jnp.maximum(h, 0.0)

    r = block_rows
    return pl.pallas_call(
        body,
        grid=(n // r,),
        in_specs=[
            pl.BlockSpec((r, fh), lambda i: (i, 0)),
            pl.BlockSpec((r, fh), lambda i: (i, 0)),
            pl.BlockSpec((r, fh), lambda i: (i, 0)),
            pl.BlockSpec((r, fh), lambda i: (i, 0)),
            pl.BlockSpec((r, 1), lambda i: (i, 0)),
            pl.BlockSpec((fh, fh), lambda i: (0, 0)),
        ],
        out_specs=pl.BlockSpec((r, fh), lambda i: (i, 0)),
        out_shape=jax.ShapeDtypeStruct((n, fh), jnp.float32),
    )(s0, s1, hp, h0, dinv, w_eff)


def _tc_final(s0, s1, hp, h0, dinv, w_eff, w_fin, bf2, block_rows):
    n, fh = hp.shape
    fo = w_fin.shape[1]

    def body(s0_ref, s1_ref, hp_ref, h0_ref, dinv_ref, w_ref, wf_ref, bf_ref,
             out_ref):
        dinv = dinv_ref[...]
        sup = ((1.0 - _ALPHA) * dinv * (s0_ref[...] + s1_ref[...] + hp_ref[...])
               + _ALPHA * h0_ref[...])
        h = jnp.dot(sup, w_ref[...], preferred_element_type=jnp.float32,
                    precision=lax.Precision.HIGHEST)
        h = jnp.maximum(h, 0.0)
        logits = jnp.dot(h, wf_ref[...], preferred_element_type=jnp.float32,
                         precision=lax.Precision.HIGHEST) + bf_ref[...]
        m = jnp.max(logits, axis=1, keepdims=True)
        z = logits - m
        lse = jnp.log(jnp.sum(jnp.exp(z), axis=1, keepdims=True))
        out_ref[...] = z - lse

    r = block_rows
    return pl.pallas_call(
        body,
        grid=(n // r,),
        in_specs=[
            pl.BlockSpec((r, fh), lambda i: (i, 0)),
            pl.BlockSpec((r, fh), lambda i: (i, 0)),
            pl.BlockSpec((r, fh), lambda i: (i, 0)),
            pl.BlockSpec((r, fh), lambda i: (i, 0)),
            pl.BlockSpec((r, 1), lambda i: (i, 0)),
            pl.BlockSpec((fh, fh), lambda i: (0, 0)),
            pl.BlockSpec((fh, fo), lambda i: (0, 0)),
            pl.BlockSpec((1, fo), lambda i: (0, 0)),
        ],
        out_specs=pl.BlockSpec((r, fo), lambda i: (i, 0)),
        out_shape=jax.ShapeDtypeStruct((n, fo), jnp.float32),
    )(s0, s1, hp, h0, dinv, w_eff, w_fin, bf2)


def kernel(x, edge_index, W_init, b_init, W_layers, W_final, b_final):
    n, _ = x.shape
    e = edge_index.shape[1]
    ew = e // _NW
    nch = ew // _K
    assert ew * _NW == e and nch * _K == ew and n % 8 == 0
    src2 = edge_index[0].reshape(_NW, ew)
    dst3 = edge_index[1].reshape(_NW, ew // 80, 80)
    block_rows = 1000
    br, _tail = _row_split(n)
    fh = W_init.shape[1]
    zeros16 = jnp.zeros((br, 16), jnp.float32)
    zerosf = jnp.zeros((br, fh), jnp.float32)

    degp = _deg_parts(dst3, zeros16, n)
    h0, hp, dinv = _tc_init(x, W_init, b_init.reshape(1, -1),
                            degp[0], degp[1], block_rows)

    num_layers = W_layers.shape[0]
    eye = jnp.eye(W_layers.shape[1], dtype=jnp.float32)
    out = None
    for i in range(num_layers):
        beta = math.log(_BETA_BASE / (i + 1) + 1.0)
        w_eff = (1.0 - beta) * eye + beta * W_layers[i]
        s = _sc_scatter(hp, src2, dst3, zerosf)
        if i + 1 < num_layers:
            hp = _tc_layer(s[0], s[1], hp, h0, dinv, w_eff, block_rows)
        else:
            out = _tc_final(s[0], s[1], hp, h0, dinv, w_eff,
                            W_final, b_final.reshape(1, -1), block_rows)
    return out


# speedup vs baseline: 22.3424x; 1.0584x over previous
import functools
import math

import jax
import jax.numpy as jnp
from jax import lax
from jax.experimental import pallas as pl
from jax.experimental.pallas import tpu as pltpu
from jax.experimental.pallas import tpu_sc as plsc

_ALPHA = 0.1
_BETA_BASE = 0.5

_NC = 2
_NS = 16
_NW = _NC * _NS
_K = 40
_NB = 4


def _sc_mesh():
    return plsc.VectorSubcoreMesh(
        core_axis_name="c", subcore_axis_name="s",
        num_cores=_NC, num_subcores=_NS)


def _row_split(n):
    br = 8 * (n // (8 * _NS))
    tail = n - br * _NS
    return br, tail


def _deg_parts(dst3, zeros16, n):
    nw, nch, k = dst3.shape
    br, tail = _row_split(n)

    @functools.partial(
        pl.kernel,
        out_type=jax.ShapeDtypeStruct((_NC, n, 16), jnp.float32),
        mesh=_sc_mesh(),
        scratch_types=[
            pltpu.VMEM((nch, k), jnp.int32),
            pltpu.VMEM((k, 16), jnp.float32),
            pltpu.VMEM_SHARED((n, 16), jnp.float32),
        ],
    )
    def kern(dst_hbm, zer_hbm, out_hbm, didx, ones_v, acc):
        cid = lax.axis_index("c")
        sid = lax.axis_index("s")
        wid = sid * _NC + cid
        pltpu.sync_copy(dst_hbm.at[wid], didx)
        onehot = jnp.where(lax.iota(jnp.int32, 16) == 0, 1.0, 0.0)

        @pl.loop(0, k)
        def _(r):
            ones_v[r, :] = onehot

        pltpu.sync_copy(zer_hbm.at[pl.ds(0, br)], acc.at[pl.ds(sid * br, br)])

        @pl.when(sid == _NS - 1)
        def _():
            pltpu.sync_copy(zer_hbm.at[pl.ds(0, tail)],
                            acc.at[pl.ds(_NS * br, tail)])

        plsc.subcore_barrier()

        @pl.loop(0, nch)
        def _(j):
            pltpu.sync_copy(ones_v, acc.at[didx.at[j]], add=True)

        plsc.subcore_barrier()
        pltpu.sync_copy(acc.at[pl.ds(sid * br, br)],
                        out_hbm.at[cid, pl.ds(sid * br, br)])

        @pl.when(sid == _NS - 1)
        def _():
            pltpu.sync_copy(acc.at[pl.ds(_NS * br, tail)],
                            out_hbm.at[cid, pl.ds(_NS * br, tail)])

    return kern(dst3, zeros16)


def _sc_scatter(hp, src2, dst3, zerosf):
    n, f = hp.shape
    ew = src2.shape[1]
    k = _K
    nch = ew // k
    br, tail = _row_split(n)

    nb = _NB
    assert nb % 2 == 0 and nch % 2 == 0
    nmain = (nch // nb) * nb

    @functools.partial(
        pl.kernel,
        out_type=jax.ShapeDtypeStruct((_NC, n, f), jnp.float32),
        mesh=_sc_mesh(),
        scratch_types=[
            pltpu.VMEM((ew,), jnp.int32),
            pltpu.VMEM((ew // 80, 80), jnp.int32),
            pltpu.VMEM((nb, k, f), jnp.float32),
            pltpu.VMEM_SHARED((n, f), jnp.float32),
        ] + [pltpu.SemaphoreType.DMA] * (2 * nb + 2),
    )
    def kern(hp_hbm, src_hbm, dst_hbm, zer_hbm, out_hbm, sidx, didx, rows, acc,
             *sems):
        gsem, ssem = sems[:nb], sems[nb:2 * nb]
        isem1, isem2 = sems[2 * nb], sems[2 * nb + 1]
        cid = lax.axis_index("c")
        sid = lax.axis_index("s")
        wid = sid * _NC + cid
        c1 = pltpu.async_copy(src_hbm.at[wid], sidx, isem1)
        c2 = pltpu.async_copy(dst_hbm.at[wid], didx, isem2)
        c3 = pltpu.async_copy(zer_hbm.at[pl.ds(0, br)],
                              acc.at[pl.ds(sid * br, br)], isem2)
        c1.wait()

        def dslice(j, b):
            return didx.at[j // 2, pl.ds((b % 2) * k, k)]

        for b in range(nb):
            pltpu.async_copy(hp_hbm.at[sidx.at[pl.ds(b * k, k)]], rows.at[b],
                             gsem[b])

        @pl.when(sid == _NS - 1)
        def _():
            pltpu.sync_copy(zer_hbm.at[pl.ds(0, tail)],
                            acc.at[pl.ds(_NS * br, tail)])

        c2.wait()
        c3.wait()
        plsc.subcore_barrier()

        @pl.loop(0, nmain, step=nb)
        def _(j0):
            for b in range(nb):
                j = j0 + b
                pltpu.make_async_copy(hp_hbm.at[sidx.at[pl.ds(j * k, k)]],
                                      rows.at[b], gsem[b]).wait()
                pltpu.async_copy(rows.at[b], acc.at[dslice(j, b)], ssem[b],
                                 add=True)
            for b in range(nb):
                j = j0 + b
                pltpu.make_async_copy(rows.at[b], acc.at[dslice(j, b)],
                                      ssem[b]).wait()
                jn = j + nb

                @pl.when(jn < nch)
                def _():
                    pltpu.async_copy(hp_hbm.at[sidx.at[pl.ds(jn * k, k)]],
                                     rows.at[b], gsem[b])

        for j in range(nmain, nch):
            b = j % nb
            pltpu.make_async_copy(hp_hbm.at[sidx.at[pl.ds(j * k, k)]],
                                  rows.at[b], gsem[b]).wait()
            pltpu.sync_copy(rows.at[b], acc.at[dslice(j, b)], add=True)

        plsc.subcore_barrier()
        pltpu.sync_copy(acc.at[pl.ds(sid * br, br)],
                        out_hbm.at[cid, pl.ds(sid * br, br)])

        @pl.when(sid == _NS - 1)
        def _():
            pltpu.sync_copy(acc.at[pl.ds(_NS * br, tail)],
                            out_hbm.at[cid, pl.ds(_NS * br, tail)])

    return kern(hp, src2, dst3, zerosf)


def _tc_init(x, w, b2, degp, block_rows):
    n, fin = x.shape
    fh = w.shape[1]

    def body(x_ref, w_ref, b_ref, d_ref, h0_ref, hp_ref, dinv_ref):
        deg = d_ref[0, :, 0:1] + d_ref[1, :, 0:1] + 1.0
        dinv = lax.rsqrt(deg)
        h = jnp.dot(x_ref[...], w_ref[...],
                    preferred_element_type=jnp.float32,
                    precision=lax.Precision.HIGHEST)
        h = jnp.maximum(h + b_ref[...], 0.0)
        h0_ref[...] = h
        hp_ref[...] = dinv * h
        dinv_ref[...] = dinv

    r = block_rows
    return pl.pallas_call(
        body,
        grid=(n // r,),
        in_specs=[
            pl.BlockSpec((r, fin), lambda i: (i, 0)),
            pl.BlockSpec((fin, fh), lambda i: (0, 0)),
            pl.BlockSpec((1, fh), lambda i: (0, 0)),
            pl.BlockSpec((2, r, 16), lambda i: (0, i, 0)),
        ],
        out_specs=[
            pl.BlockSpec((r, fh), lambda i: (i, 0)),
            pl.BlockSpec((r, fh), lambda i: (i, 0)),
            pl.BlockSpec((r, 1), lambda i: (i, 0)),
        ],
        out_shape=[
            jax.ShapeDtypeStruct((n, fh), jnp.float32),
            jax.ShapeDtypeStruct((n, fh), jnp.float32),
            jax.ShapeDtypeStruct((n, 1), jnp.float32),
        ],
    )(x, w, b2, degp)


def _tc_layer(s, hp, h0, dinv, w_eff, block_rows):
    n, fh = hp.shape

    def body(s_ref, hp_ref, h0_ref, dinv_ref, w_ref, out_ref):
        dinv = dinv_ref[...]
        sup = ((1.0 - _ALPHA) * dinv * (s_ref[0] + s_ref[1] + hp_ref[...])
               + _ALPHA * h0_ref[...])
        h = jnp.dot(sup, w_ref[...], preferred_element_type=jnp.float32,
                    precision=lax.Precision.HIGHEST)
        out_ref[...] = dinv * jnp.maximum(h, 0.0)

    r = block_rows
    return pl.pallas_call(
        body,
        grid=(n // r,),
        in_specs=[
            pl.BlockSpec((2, r, fh), lambda i: (0, i, 0)),
            pl.BlockSpec((r, fh), lambda i: (i, 0)),
            pl.BlockSpec((r, fh), lambda i: (i, 0)),
            pl.BlockSpec((r, 1), lambda i: (i, 0)),
            pl.BlockSpec((fh, fh), lambda i: (0, 0)),
        ],
        out_specs=pl.BlockSpec((r, fh), lambda i: (i, 0)),
        out_shape=jax.ShapeDtypeStruct((n, fh), jnp.float32),
    )(s, hp, h0, dinv, w_eff)


def _tc_final(s, hp, h0, dinv, w_eff, w_fin, bf2, block_rows):
    n, fh = hp.shape
    fo = w_fin.shape[1]

    def body(s_ref, hp_ref, h0_ref, dinv_ref, w_ref, wf_ref, bf_ref,
             out_ref):
        dinv = dinv_ref[...]
        sup = ((1.0 - _ALPHA) * dinv * (s_ref[0] + s_ref[1] + hp_ref[...])
               + _ALPHA * h0_ref[...])
        h = jnp.dot(sup, w_ref[...], preferred_element_type=jnp.float32,
                    precision=lax.Precision.HIGHEST)
        h = jnp.maximum(h, 0.0)
        logits = jnp.dot(h, wf_ref[...], preferred_element_type=jnp.float32,
                         precision=lax.Precision.HIGHEST) + bf_ref[...]
        m = jnp.max(logits, axis=1, keepdims=True)
        z = logits - m
        lse = jnp.log(jnp.sum(jnp.exp(z), axis=1, keepdims=True))
        out_ref[...] = z - lse

    r = block_rows
    return pl.pallas_call(
        body,
        grid=(n // r,),
        in_specs=[
            pl.BlockSpec((2, r, fh), lambda i: (0, i, 0)),
            pl.BlockSpec((r, fh), lambda i: (i, 0)),
            pl.BlockSpec((r, fh), lambda i: (i, 0)),
            pl.BlockSpec((r, 1), lambda i: (i, 0)),
            pl.BlockSpec((fh, fh), lambda i: (0, 0)),
            pl.BlockSpec((fh, fo), lambda i: (0, 0)),
            pl.BlockSpec((1, fo), lambda i: (0, 0)),
        ],
        out_specs=pl.BlockSpec((r, fo), lambda i: (i, 0)),
        out_shape=jax.ShapeDtypeStruct((n, fo), jnp.float32),
    )(s, hp, h0, dinv, w_eff, w_fin, bf2)


def kernel(x, edge_index, W_init, b_init, W_layers, W_final, b_final):
    n, _ = x.shape
    e = edge_index.shape[1]
    ew = e // _NW
    nch = ew // _K
    assert ew * _NW == e and nch * _K == ew and n % 8 == 0
    src2 = edge_index[0].reshape(_NW, ew)
    dst3 = edge_index[1].reshape(_NW, ew // 80, 80)
    block_rows = 1000
    br, _tail = _row_split(n)
    fh = W_init.shape[1]
    zeros16 = jnp.zeros((br, 16), jnp.float32)
    zerosf = jnp.zeros((br, fh), jnp.float32)

    degp = _deg_parts(dst3, zeros16, n)
    h0, hp, dinv = _tc_init(x, W_init, b_init.reshape(1, -1),
                            degp, block_rows)

    num_layers = W_layers.shape[0]
    eye = jnp.eye(W_layers.shape[1], dtype=jnp.float32)
    out = None
    for i in range(num_layers):
        beta = math.log(_BETA_BASE / (i + 1) + 1.0)
        w_eff = (1.0 - beta) * eye + beta * W_layers[i]
        s = _sc_scatter(hp, src2, dst3, zerosf)
        if i + 1 < num_layers:
            hp = _tc_layer(s, hp, h0, dinv, w_eff, block_rows)
        else:
            out = _tc_final(s, hp, h0, dinv, w_eff,
                            W_final, b_final.reshape(1, -1), block_rows)
    return out


# speedup vs baseline: 22.4994x; 1.0070x over previous
import functools
import math

import jax
import jax.numpy as jnp
from jax import lax
from jax.experimental import pallas as pl
from jax.experimental.pallas import tpu as pltpu
from jax.experimental.pallas import tpu_sc as plsc

_ALPHA = 0.1
_BETA_BASE = 0.5

_NC = 2
_NS = 16
_NW = _NC * _NS
_K = 40
_NB = 4


def _sc_mesh():
    return plsc.VectorSubcoreMesh(
        core_axis_name="c", subcore_axis_name="s",
        num_cores=_NC, num_subcores=_NS)


def _row_split(n):
    br = 8 * (n // (8 * _NS))
    tail = n - br * _NS
    return br, tail


def _deg_parts(dst3, zeros16, n):
    nw, nch, k = dst3.shape
    br, tail = _row_split(n)

    @functools.partial(
        pl.kernel,
        out_type=jax.ShapeDtypeStruct((_NC * n, 16), jnp.float32),
        mesh=_sc_mesh(),
        scratch_types=[
            pltpu.VMEM((nch, k), jnp.int32),
            pltpu.VMEM((k, 16), jnp.float32),
            pltpu.VMEM_SHARED((n, 16), jnp.float32),
        ] + [pltpu.SemaphoreType.DMA] * 4,
    )
    def kern(dst_hbm, zer_hbm, out_hbm, didx, ones_v, acc, *dsem):
        cid = lax.axis_index("c")
        sid = lax.axis_index("s")
        wid = sid * _NC + cid
        pltpu.sync_copy(dst_hbm.at[wid], didx)
        onehot = jnp.where(lax.iota(jnp.int32, 16) == 0, 1.0, 0.0)

        @pl.loop(0, k)
        def _(r):
            ones_v[r, :] = onehot

        pltpu.sync_copy(zer_hbm.at[pl.ds(0, br)], acc.at[pl.ds(sid * br, br)])

        @pl.when(sid == _NS - 1)
        def _():
            pltpu.sync_copy(zer_hbm.at[pl.ds(0, tail)],
                            acc.at[pl.ds(_NS * br, tail)])

        plsc.subcore_barrier()
        nmain = (nch // 4) * 4

        @pl.loop(0, nmain, step=4)
        def _(j0):
            for b in range(4):
                pltpu.async_copy(ones_v, acc.at[didx.at[j0 + b]], dsem[b],
                                 add=True)
            for b in range(4):
                pltpu.make_async_copy(ones_v, acc.at[didx.at[j0 + b]],
                                      dsem[b]).wait()

        for j in range(nmain, nch):
            pltpu.sync_copy(ones_v, acc.at[didx.at[j]], add=True)

        plsc.subcore_barrier()
        pltpu.sync_copy(acc.at[pl.ds(sid * br, br)],
                        out_hbm.at[pl.ds(cid * n + sid * br, br)])

        @pl.when(sid == _NS - 1)
        def _():
            pltpu.sync_copy(acc.at[pl.ds(_NS * br, tail)],
                            out_hbm.at[pl.ds(cid * n + _NS * br, tail)])

    return kern(dst3, zeros16)


def _sc_scatter(hp, src2, dst3, zerosf):
    n, f = hp.shape
    ew = src2.shape[1]
    k = _K
    nch = ew // k
    br, tail = _row_split(n)

    nb = _NB
    assert nb % 2 == 0 and nch % 2 == 0
    nmain = (nch // nb) * nb

    @functools.partial(
        pl.kernel,
        out_type=jax.ShapeDtypeStruct((_NC * n, f), jnp.float32),
        mesh=_sc_mesh(),
        scratch_types=[
            pltpu.VMEM((ew,), jnp.int32),
            pltpu.VMEM((ew // 80, 80), jnp.int32),
            pltpu.VMEM((nb, k, f), jnp.float32),
            pltpu.VMEM_SHARED((n, f), jnp.float32),
        ] + [pltpu.SemaphoreType.DMA] * (2 * nb + 2),
    )
    def kern(hp_hbm, src_hbm, dst_hbm, zer_hbm, out_hbm, sidx, didx, rows, acc,
             *sems):
        gsem, ssem = sems[:nb], sems[nb:2 * nb]
        isem1, isem2 = sems[2 * nb], sems[2 * nb + 1]
        cid = lax.axis_index("c")
        sid = lax.axis_index("s")
        wid = sid * _NC + cid
        c1 = pltpu.async_copy(src_hbm.at[wid], sidx, isem1)
        c2 = pltpu.async_copy(dst_hbm.at[wid], didx, isem2)
        c3 = pltpu.async_copy(zer_hbm.at[pl.ds(0, br)],
                              acc.at[pl.ds(sid * br, br)], isem2)
        c1.wait()

        def dslice(j, b):
            return didx.at[j // 2, pl.ds((b % 2) * k, k)]

        for b in range(nb):
            pltpu.async_copy(hp_hbm.at[sidx.at[pl.ds(b * k, k)]], rows.at[b],
                             gsem[b])

        @pl.when(sid == _NS - 1)
        def _():
            pltpu.sync_copy(zer_hbm.at[pl.ds(0, tail)],
                            acc.at[pl.ds(_NS * br, tail)])

        c2.wait()
        c3.wait()
        plsc.subcore_barrier()

        @pl.loop(0, nmain, step=nb)
        def _(j0):
            for b in range(nb):
                j = j0 + b
                pltpu.make_async_copy(hp_hbm.at[sidx.at[pl.ds(j * k, k)]],
                                      rows.at[b], gsem[b]).wait()
                pltpu.async_copy(rows.at[b], acc.at[dslice(j, b)], ssem[b],
                                 add=True)
            for b in range(nb):
                j = j0 + b
                pltpu.make_async_copy(rows.at[b], acc.at[dslice(j, b)],
                                      ssem[b]).wait()
                jn = j + nb

                @pl.when(jn < nch)
                def _():
                    pltpu.async_copy(hp_hbm.at[sidx.at[pl.ds(jn * k, k)]],
                                     rows.at[b], gsem[b])

        for j in range(nmain, nch):
            b = j % nb
            pltpu.make_async_copy(hp_hbm.at[sidx.at[pl.ds(j * k, k)]],
                                  rows.at[b], gsem[b]).wait()
            pltpu.sync_copy(rows.at[b], acc.at[dslice(j, b)], add=True)

        plsc.subcore_barrier()
        pltpu.sync_copy(acc.at[pl.ds(sid * br, br)],
                        out_hbm.at[pl.ds(cid * n + sid * br, br)])

        @pl.when(sid == _NS - 1)
        def _():
            pltpu.sync_copy(acc.at[pl.ds(_NS * br, tail)],
                            out_hbm.at[pl.ds(cid * n + _NS * br, tail)])

    return kern(hp, src2, dst3, zerosf)


def _tc_init(x, w, b2, degp, block_rows):
    n, fin = x.shape
    fh = w.shape[1]

    def body(x_ref, w_ref, b_ref, d0_ref, d1_ref, h0_ref, hp_ref,
             dinv_ref):
        deg = d0_ref[:, 0:1] + d1_ref[:, 0:1] + 1.0
        dinv = lax.rsqrt(deg)
        h = jnp.dot(x_ref[...], w_ref[...],
                    preferred_element_type=jnp.float32,
                    precision=lax.Precision.HIGHEST)
        h = jnp.maximum(h + b_ref[...], 0.0)
        h0_ref[...] = h
        hp_ref[...] = dinv * h
        dinv_ref[...] = dinv

    r = block_rows
    return pl.pallas_call(
        body,
        grid=(n // r,),
        in_specs=[
            pl.BlockSpec((r, fin), lambda i: (i, 0)),
            pl.BlockSpec((fin, fh), lambda i: (0, 0)),
            pl.BlockSpec((1, fh), lambda i: (0, 0)),
            pl.BlockSpec((r, 16), lambda i: (i, 0)),
            pl.BlockSpec((r, 16), lambda i, _nb=n // block_rows: (i + _nb, 0)),
        ],
        out_specs=[
            pl.BlockSpec((r, fh), lambda i: (i, 0)),
            pl.BlockSpec((r, fh), lambda i: (i, 0)),
            pl.BlockSpec((r, 1), lambda i: (i, 0)),
        ],
        out_shape=[
            jax.ShapeDtypeStruct((n, fh), jnp.float32),
            jax.ShapeDtypeStruct((n, fh), jnp.float32),
            jax.ShapeDtypeStruct((n, 1), jnp.float32),
        ],
    )(x, w, b2, degp, degp)


def _tc_layer(s, hp, h0, dinv, w_eff, block_rows):
    n, fh = hp.shape

    def body(s0_ref, s1_ref, hp_ref, h0_ref, dinv_ref, w_ref, out_ref):
        dinv = dinv_ref[...]
        sup = ((1.0 - _ALPHA) * dinv
               * (s0_ref[...] + s1_ref[...] + hp_ref[...])
               + _ALPHA * h0_ref[...])
        h = jnp.dot(sup, w_ref[...], preferred_element_type=jnp.float32,
                    precision=lax.Precision.HIGHEST)
        out_ref[...] = dinv * jnp.maximum(h, 0.0)

    r = block_rows
    return pl.pallas_call(
        body,
        grid=(n // r,),
        in_specs=[
            pl.BlockSpec((r, fh), lambda i: (i, 0)),
            pl.BlockSpec((r, fh), lambda i, _nb=n // block_rows: (i + _nb, 0)),
            pl.BlockSpec((r, fh), lambda i: (i, 0)),
            pl.BlockSpec((r, fh), lambda i: (i, 0)),
            pl.BlockSpec((r, 1), lambda i: (i, 0)),
            pl.BlockSpec((fh, fh), lambda i: (0, 0)),
        ],
        out_specs=pl.BlockSpec((r, fh), lambda i: (i, 0)),
        out_shape=jax.ShapeDtypeStruct((n, fh), jnp.float32),
    )(s, s, hp, h0, dinv, w_eff)


def _tc_final(s, hp, h0, dinv, w_eff, w_fin, bf2, block_rows):
    n, fh = hp.shape
    fo = w_fin.shape[1]

    def body(s0_ref, s1_ref, hp_ref, h0_ref, dinv_ref, w_ref, wf_ref,
             bf_ref, out_ref):
        dinv = dinv_ref[...]
        sup = ((1.0 - _ALPHA) * dinv
               * (s0_ref[...] + s1_ref[...] + hp_ref[...])
               + _ALPHA * h0_ref[...])
        h = jnp.dot(sup, w_ref[...], preferred_element_type=jnp.float32,
                    precision=lax.Precision.HIGHEST)
        h = jnp.maximum(h, 0.0)
        logits = jnp.dot(h, wf_ref[...], preferred_element_type=jnp.float32,
                         precision=lax.Precision.HIGHEST) + bf_ref[...]
        m = jnp.max(logits, axis=1, keepdims=True)
        z = logits - m
        lse = jnp.log(jnp.sum(jnp.exp(z), axis=1, keepdims=True))
        out_ref[...] = z - lse

    r = block_rows
    return pl.pallas_call(
        body,
        grid=(n // r,),
        in_specs=[
            pl.BlockSpec((r, fh), lambda i: (i, 0)),
            pl.BlockSpec((r, fh), lambda i, _nb=n // block_rows: (i + _nb, 0)),
            pl.BlockSpec((r, fh), lambda i: (i, 0)),
            pl.BlockSpec((r, fh), lambda i: (i, 0)),
            pl.BlockSpec((r, 1), lambda i: (i, 0)),
            pl.BlockSpec((fh, fh), lambda i: (0, 0)),
            pl.BlockSpec((fh, fo), lambda i: (0, 0)),
            pl.BlockSpec((1, fo), lambda i: (0, 0)),
        ],
        out_specs=pl.BlockSpec((r, fo), lambda i: (i, 0)),
        out_shape=jax.ShapeDtypeStruct((n, fo), jnp.float32),
    )(s, s, hp, h0, dinv, w_eff, w_fin, bf2)


def kernel(x, edge_index, W_init, b_init, W_layers, W_final, b_final):
    n, _ = x.shape
    e = edge_index.shape[1]
    ew = e // _NW
    nch = ew // _K
    assert ew * _NW == e and nch * _K == ew and n % 8 == 0
    src2 = edge_index[0].reshape(_NW, ew)
    dst3 = edge_index[1].reshape(_NW, ew // 80, 80)
    block_rows = 1000
    br, _tail = _row_split(n)
    fh = W_init.shape[1]
    zeros16 = jnp.zeros((br, 16), jnp.float32)
    zerosf = jnp.zeros((br, fh), jnp.float32)

    degp = _deg_parts(dst3, zeros16, n)
    h0, hp, dinv = _tc_init(x, W_init, b_init.reshape(1, -1),
                            degp, block_rows)

    num_layers = W_layers.shape[0]
    eye = jnp.eye(W_layers.shape[1], dtype=jnp.float32)
    out = None
    for i in range(num_layers):
        beta = math.log(_BETA_BASE / (i + 1) + 1.0)
        w_eff = (1.0 - beta) * eye + beta * W_layers[i]
        s = _sc_scatter(hp, src2, dst3, zerosf)
        if i + 1 < num_layers:
            hp = _tc_layer(s, hp, h0, dinv, w_eff, block_rows)
        else:
            out = _tc_final(s, hp, h0, dinv, w_eff,
                            W_final, b_final.reshape(1, -1), block_rows)
    return out
